# baseline (speedup 1.0000x reference)
import jax
import jax.numpy as jnp
from jax.experimental import pallas as pl
from jax.experimental.pallas import tpu as pltpu

_STEPS = 100
_BR = 8
_N = 4


def _make_body(nsteps):
    def _body(t_ref, sac_ref, somac_ref, x_hbm, n_hbm, o_hbm,
              xb, nb, ob, in_sems, out_sems):
        i = pl.program_id(0)

        def in_copies(j, slot):
            cx = pltpu.make_async_copy(
                x_hbm.at[pl.ds(j * _BR, _BR), :], xb.at[slot],
                in_sems.at[slot, 0])
            cn = pltpu.make_async_copy(
                n_hbm.at[pl.ds(j * _BR, _BR), :], nb.at[slot],
                in_sems.at[slot, 1])
            return cx, cn

        def out_copy(j, slot):
            return pltpu.make_async_copy(
                ob.at[slot], o_hbm.at[pl.ds(j * _BR, _BR), :],
                out_sems.at[slot])

        @pl.when(i == 0)
        def _warmup():
            for j in range(_N):
                cx, cn = in_copies(j, j)
                cx.start()
                cn.start()

        slot = jax.lax.rem(i, _N)
        cx, cn = in_copies(i, slot)
        cx.wait()
        cn.wait()

        @pl.when(i >= _N)
        def _wait_prev_out():
            out_copy(i - _N, slot).wait()

        tcol = t_ref[pl.ds(i * _BR, _BR), :]
        steps = jax.lax.broadcasted_iota(jnp.int32, (_BR, _STEPS), 1)
        m = tcol == steps
        c1 = jnp.sum(jnp.where(m, sac_ref[...], 0.0), axis=1, keepdims=True)
        c2 = jnp.sum(jnp.where(m, somac_ref[...], 0.0), axis=1, keepdims=True)
        ob[slot] = c1 * xb[slot] + c2 * nb[slot]

        out_copy(i, slot).start()

        @pl.when(i + _N < nsteps)
        def _prefetch():
            cx2, cn2 = in_copies(i + _N, slot)
            cx2.start()
            cn2.start()

        @pl.when(i == nsteps - 1)
        def _drain():
            for k in range(_N):
                j = nsteps - _N + k
                out_copy(j, j % _N).wait()

    return _body


def kernel(x_start, noise, sqrt_alphas_cumprod, sqrt_one_minus_alphas_cumprod, t):
    B, V = x_start.shape
    nsteps = B // _BR
    t2 = t.reshape(B, 1)
    sac2 = sqrt_alphas_cumprod.reshape(1, _STEPS)
    somac2 = sqrt_one_minus_alphas_cumprod.reshape(1, _STEPS)

    hbm = pl.BlockSpec(memory_space=pltpu.MemorySpace.HBM)
    return pl.pallas_call(
        _make_body(nsteps),
        grid=(nsteps,),
        in_specs=[
            pl.BlockSpec((B, 1), lambda i: (0, 0)),
            pl.BlockSpec((1, _STEPS), lambda i: (0, 0)),
            pl.BlockSpec((1, _STEPS), lambda i: (0, 0)),
            hbm,
            hbm,
        ],
        out_specs=pl.BlockSpec(memory_space=pltpu.MemorySpace.HBM),
        out_shape=jax.ShapeDtypeStruct((B, V), x_start.dtype),
        scratch_shapes=[
            pltpu.VMEM((_N, _BR, V), jnp.float32),
            pltpu.VMEM((_N, _BR, V), jnp.float32),
            pltpu.VMEM((_N, _BR, V), jnp.float32),
            pltpu.SemaphoreType.DMA((_N, 2)),
            pltpu.SemaphoreType.DMA((_N,)),
        ],
        compiler_params=pltpu.CompilerParams(
            dimension_semantics=("arbitrary",),
            vmem_limit_bytes=100 * 1024 * 1024,
        ),
    )(t2, sac2, somac2, x_start, noise)
